# baseline (speedup 1.0000x reference)
import functools

import jax
import jax.numpy as jnp
from jax import lax
from jax.experimental import pallas as pl
from jax.experimental.pallas import tpu as pltpu
from jax.experimental.pallas import tpu_sc as plsc

E = 8
K = 2
D = 1024
F = 2048
TOKS = 4096
A = TOKS * K
BLK = 256
F2 = F // 2
G = A // BLK + E
R = G * BLK
NC, NS = 2, 16
NW = NC * NS

_CONTRACT_MINOR = (((1,), (1,)), ((), ()))


def _router_body(x_ref, w_ref, ti_ref, tw_ref):
    xg = x_ref[...]
    logits = lax.dot_general(xg, w_ref[...], _CONTRACT_MINOR,
                             preferred_element_type=jnp.float32)
    m = jnp.max(logits, axis=1, keepdims=True)
    p = jnp.exp(logits - m)
    probs = p / jnp.sum(p, axis=1, keepdims=True)
    iota = lax.broadcasted_iota(jnp.int32, probs.shape, 1)
    m1 = jnp.max(probs, axis=1, keepdims=True)
    i1 = jnp.min(jnp.where(probs == m1, iota, E), axis=1, keepdims=True)
    probs2 = jnp.where(iota == i1, -1.0, probs)
    m2 = jnp.max(probs2, axis=1, keepdims=True)
    i2 = jnp.min(jnp.where(probs2 == m2, iota, E), axis=1, keepdims=True)
    sw = m1 + m2
    ti_ref[:, 0:1] = i1
    ti_ref[:, 1:2] = i2
    tw_ref[:, 0:1] = m1 / sw
    tw_ref[:, 1:2] = m2 / sw


def _router(xf, w_router):
    bt = TOKS // 4
    return pl.pallas_call(
        _router_body,
        grid=(4,),
        in_specs=[pl.BlockSpec((bt, D), lambda g: (g, 0)),
                  pl.BlockSpec((E, D), lambda g: (0, 0))],
        out_specs=[pl.BlockSpec((bt, K), lambda g: (g, 0)),
                   pl.BlockSpec((bt, K), lambda g: (g, 0))],
        out_shape=[jax.ShapeDtypeStruct((TOKS, K), jnp.int32),
                   jax.ShapeDtypeStruct((TOKS, K), jnp.float32)],
    )(xf, w_router)


def _sc_mesh():
    return plsc.VectorSubcoreMesh(core_axis_name="c", subcore_axis_name="s",
                                  num_cores=NC, num_subcores=NS)


def _dispatch(xf, src3, dst3, nch, ch):
    @functools.partial(
        pl.kernel,
        out_type=jax.ShapeDtypeStruct((R, D), jnp.float32),
        mesh=_sc_mesh(),
        scratch_types=[pltpu.VMEM((nch, ch), jnp.int32),
                       pltpu.VMEM((nch, ch), jnp.int32),
                       pltpu.VMEM((2, ch, D), jnp.float32),
                       pltpu.SemaphoreType.DMA,
                       pltpu.SemaphoreType.DMA],
    )
    def disp_k(x_hbm, src_hbm, dst_hbm, out_hbm, src_v, dst_v, rows_v,
               gsem, wsem):
        wid = lax.axis_index("s") * NC + lax.axis_index("c")
        pltpu.sync_copy(src_hbm.at[wid], src_v)
        pltpu.sync_copy(dst_hbm.at[wid], dst_v)
        gh = [None, None]
        wh = [None, None]
        for i in range(nch):
            b = i % 2
            if i >= 2:
                wh[b].wait()
            gh[b] = pltpu.async_copy(x_hbm.at[src_v.at[i]], rows_v.at[b],
                                     gsem)
            if i >= 1:
                pb = (i - 1) % 2
                gh[pb].wait()
                wh[pb] = pltpu.async_copy(rows_v.at[pb],
                                          out_hbm.at[dst_v.at[i - 1]], wsem)
        lb = (nch - 1) % 2
        gh[lb].wait()
        pltpu.async_copy(rows_v.at[lb], out_hbm.at[dst_v.at[nch - 1]],
                         wsem).wait()
        if nch >= 2:
            wh[(nch - 2) % 2].wait()

    return disp_k(xf, src3, dst3)


def _unsort(out_sorted, idx3, nch, ch):
    per_w = nch * ch

    @functools.partial(
        pl.kernel,
        out_type=jax.ShapeDtypeStruct((A, D), jnp.float32),
        mesh=_sc_mesh(),
        scratch_types=[pltpu.VMEM((nch, ch), jnp.int32),
                       pltpu.VMEM((2, ch, D), jnp.float32),
                       pltpu.SemaphoreType.DMA,
                       pltpu.SemaphoreType.DMA],
    )
    def unsort_k(table_hbm, idx_hbm, out_hbm, idx_v, rows_v, gsem, wsem):
        wid = lax.axis_index("s") * NC + lax.axis_index("c")
        base = wid * per_w
        pltpu.sync_copy(idx_hbm.at[wid], idx_v)
        gh = [None, None]
        wh = [None, None]
        for i in range(nch):
            b = i % 2
            if i >= 2:
                wh[b].wait()
            gh[b] = pltpu.async_copy(table_hbm.at[idx_v.at[i]], rows_v.at[b],
                                     gsem)
            if i >= 1:
                pb = (i - 1) % 2
                gh[pb].wait()
                wh[pb] = pltpu.async_copy(
                    rows_v.at[pb],
                    out_hbm.at[pl.ds(base + (i - 1) * ch, ch)], wsem)
        lb = (nch - 1) % 2
        gh[lb].wait()
        pltpu.sync_copy(rows_v.at[lb],
                        out_hbm.at[pl.ds(base + (nch - 1) * ch, ch)])
        if nch >= 2:
            wh[(nch - 2) % 2].wait()

    return unsort_k(out_sorted, idx3)


def _grouped_body(eid_ref, nv_ref, x_ref, g_ref, u_ref, d_ref, o_ref,
                  wg_s, wu_s):
    g = pl.program_id(0)
    fh = pl.program_id(1)

    @pl.when(nv_ref[g] > 0)
    def _():
        prev = eid_ref[jnp.maximum(g - 1, 0)]
        changed = jnp.logical_or(g == 0, eid_ref[g] != prev)

        @pl.when(changed)
        def _():
            wg_s[pl.ds(fh * F2, F2), :] = g_ref[0].astype(jnp.bfloat16)
            wu_s[pl.ds(fh * F2, F2), :] = u_ref[0].astype(jnp.bfloat16)

        xg = x_ref[...].astype(jnp.bfloat16)
        gg = lax.dot_general(xg, wg_s[pl.ds(fh * F2, F2), :],
                             _CONTRACT_MINOR,
                             preferred_element_type=jnp.float32)
        uu = lax.dot_general(xg, wu_s[pl.ds(fh * F2, F2), :],
                             _CONTRACT_MINOR,
                             preferred_element_type=jnp.float32)
        h = (gg * lax.logistic(gg) * uu).astype(jnp.bfloat16)
        part = lax.dot_general(h, d_ref[0].astype(jnp.bfloat16),
                               _CONTRACT_MINOR,
                               preferred_element_type=jnp.float32)

        @pl.when(fh == 0)
        def _():
            o_ref[...] = part

        @pl.when(fh == 1)
        def _():
            o_ref[...] += part


def _grouped(x_sorted, gate_w, up_w, down_w, eid, nvalid):
    grid_spec = pltpu.PrefetchScalarGridSpec(
        num_scalar_prefetch=2,
        grid=(G, 2),
        in_specs=[
            pl.BlockSpec((BLK, D), lambda g, f, e, nv: (g, 0)),
            pl.BlockSpec((1, F2, D), lambda g, f, e, nv: (e[g], f, 0)),
            pl.BlockSpec((1, F2, D), lambda g, f, e, nv: (e[g], f, 0)),
            pl.BlockSpec((1, D, F2), lambda g, f, e, nv: (e[g], 0, f)),
        ],
        out_specs=pl.BlockSpec((BLK, D), lambda g, f, e, nv: (g, 0)),
        scratch_shapes=[pltpu.VMEM((F, D), jnp.bfloat16),
                        pltpu.VMEM((F, D), jnp.bfloat16)],
    )
    return pl.pallas_call(
        _grouped_body,
        grid_spec=grid_spec,
        out_shape=jax.ShapeDtypeStruct((R, D), jnp.float32),
    )(eid, nvalid, x_sorted, gate_w, up_w, down_w)


def _shared_body(x_ref, g_ref, u_ref, d_ref, o_ref, wg_s, wu_s, wd_s):
    @pl.when(pl.program_id(0) == 0)
    def _():
        wg_s[...] = g_ref[...].astype(jnp.bfloat16)
        wu_s[...] = u_ref[...].astype(jnp.bfloat16)
        wd_s[...] = d_ref[...].astype(jnp.bfloat16)

    xg = x_ref[...].astype(jnp.bfloat16)
    gg = lax.dot_general(xg, wg_s[...], _CONTRACT_MINOR,
                         preferred_element_type=jnp.float32)
    uu = lax.dot_general(xg, wu_s[...], _CONTRACT_MINOR,
                         preferred_element_type=jnp.float32)
    h = (gg * lax.logistic(gg) * uu).astype(jnp.bfloat16)
    o_ref[...] = lax.dot_general(h, wd_s[...], _CONTRACT_MINOR,
                                 preferred_element_type=jnp.float32)


def _shared(xf, gate_w, up_w, down_w):
    bt = 512
    return pl.pallas_call(
        _shared_body,
        grid=(TOKS // bt,),
        in_specs=[pl.BlockSpec((bt, D), lambda g: (g, 0)),
                  pl.BlockSpec((F, D), lambda g: (0, 0)),
                  pl.BlockSpec((F, D), lambda g: (0, 0)),
                  pl.BlockSpec((D, F), lambda g: (0, 0))],
        out_specs=pl.BlockSpec((bt, D), lambda g: (g, 0)),
        out_shape=jax.ShapeDtypeStruct((TOKS, D), jnp.float32),
        scratch_shapes=[pltpu.VMEM((F, D), jnp.bfloat16),
                        pltpu.VMEM((F, D), jnp.bfloat16),
                        pltpu.VMEM((D, F), jnp.bfloat16)],
    )(xf, gate_w, up_w, down_w)


def _combine_body(sh_ref, p0_ref, p1_ref, tw_ref, o_ref):
    w0 = tw_ref[:, 0:1]
    w1 = tw_ref[:, 1:2]
    o_ref[...] = sh_ref[...] + w0 * p0_ref[...] + w1 * p1_ref[...]


def _combine(sh, pairs, top_w):
    bt = 1024
    nb = TOKS // bt
    return pl.pallas_call(
        _combine_body,
        grid=(nb,),
        in_specs=[pl.BlockSpec((bt, D), lambda g: (g, 0)),
                  pl.BlockSpec((bt, D), lambda g: (g, 0)),
                  pl.BlockSpec((bt, D), lambda g: (g + nb, 0)),
                  pl.BlockSpec((bt, K), lambda g: (g, 0))],
        out_specs=pl.BlockSpec((bt, D), lambda g: (g, 0)),
        out_shape=jax.ShapeDtypeStruct((TOKS, D), jnp.float32),
    )(sh, pairs, pairs, top_w)


def kernel(x, w_router, shared_gate, shared_up, shared_down,
           experts_gate, experts_up, experts_down):
    b, s, d = x.shape
    xf = x.reshape(-1, d)

    sh = _shared(xf, shared_gate, shared_up, shared_down)

    top_i, top_w = _router(xf, w_router)

    ae = top_i.reshape(A)
    onehot = (ae[:, None] == jnp.arange(E, dtype=jnp.int32)[None, :])
    ranks = jnp.cumsum(onehot.astype(jnp.int32), axis=0)
    counts = ranks[-1]
    rank = jnp.take_along_axis(ranks, ae[:, None], axis=1)[:, 0] - 1
    padded = ((counts + BLK - 1) // BLK) * BLK
    ends = jnp.cumsum(padded).astype(jnp.int32)
    offs = ends - padded
    dest = (offs[ae] + rank).astype(jnp.int32)

    gstart = jnp.arange(G, dtype=jnp.int32) * BLK
    eid = jnp.sum((gstart[:, None] >= ends[None, :]).astype(jnp.int32),
                  axis=1)
    eid_c = jnp.minimum(eid, E - 1)
    nvalid = jnp.where(
        eid < E,
        jnp.clip(offs[eid_c] + counts[eid_c] - gstart, 0, BLK),
        0).astype(jnp.int32)

    nch1, ch1 = 8, A // NW // 8
    src3 = (jnp.arange(A, dtype=jnp.int32) // K).reshape(NW, nch1, ch1)
    dst3 = dest.reshape(NW, nch1, ch1)
    x_sorted = _dispatch(xf, src3, dst3, nch1, ch1)

    out_sorted = _grouped(x_sorted, experts_gate, experts_up, experts_down,
                          eid_c, nvalid)

    idx_sm = dest.reshape(TOKS, K).T.reshape(NW, nch1, ch1)
    pairs = _unsort(out_sorted, idx_sm, nch1, ch1)

    out = _combine(sh, pairs, top_w)
    return out.reshape(b, s, d)


# speedup vs baseline: 1.3682x; 1.3682x over previous
import functools

import jax
import jax.numpy as jnp
from jax import lax
from jax.experimental import pallas as pl
from jax.experimental.pallas import tpu as pltpu
from jax.experimental.pallas import tpu_sc as plsc

E = 8
K = 2
D = 1024
F = 2048
TOKS = 4096
A = TOKS * K
BLK = 256
G = A // BLK + E
R = G * BLK
NC, NS = 2, 16
NW = NC * NS

_CONTRACT_MINOR = (((1,), (1,)), ((), ()))


def _router_body(x_ref, w_ref, ti_ref, tw_ref):
    xg = x_ref[...]
    logits = lax.dot_general(xg, w_ref[...], _CONTRACT_MINOR,
                             preferred_element_type=jnp.float32)
    m = jnp.max(logits, axis=1, keepdims=True)
    p = jnp.exp(logits - m)
    probs = p / jnp.sum(p, axis=1, keepdims=True)
    iota = lax.broadcasted_iota(jnp.int32, probs.shape, 1)
    m1 = jnp.max(probs, axis=1, keepdims=True)
    i1 = jnp.min(jnp.where(probs == m1, iota, E), axis=1, keepdims=True)
    probs2 = jnp.where(iota == i1, -1.0, probs)
    m2 = jnp.max(probs2, axis=1, keepdims=True)
    i2 = jnp.min(jnp.where(probs2 == m2, iota, E), axis=1, keepdims=True)
    sw = m1 + m2
    ti_ref[:, 0:1] = i1
    ti_ref[:, 1:2] = i2
    tw_ref[:, 0:1] = m1 / sw
    tw_ref[:, 1:2] = m2 / sw


def _router(xf, w_router):
    bt = TOKS // 4
    return pl.pallas_call(
        _router_body,
        grid=(4,),
        in_specs=[pl.BlockSpec((bt, D), lambda g: (g, 0)),
                  pl.BlockSpec((E, D), lambda g: (0, 0))],
        out_specs=[pl.BlockSpec((bt, K), lambda g: (g, 0)),
                   pl.BlockSpec((bt, K), lambda g: (g, 0))],
        out_shape=[jax.ShapeDtypeStruct((TOKS, K), jnp.int32),
                   jax.ShapeDtypeStruct((TOKS, K), jnp.float32)],
    )(xf, w_router)


def _sc_mesh():
    return plsc.VectorSubcoreMesh(core_axis_name="c", subcore_axis_name="s",
                                  num_cores=NC, num_subcores=NS)


def _dispatch(xf, src3, dst3, nch, ch):
    @functools.partial(
        pl.kernel,
        out_type=jax.ShapeDtypeStruct((R, D), jnp.float32),
        mesh=_sc_mesh(),
        scratch_types=[pltpu.VMEM((nch, ch), jnp.int32),
                       pltpu.VMEM((nch, ch), jnp.int32),
                       pltpu.VMEM((2, ch, D), jnp.float32),
                       pltpu.SemaphoreType.DMA,
                       pltpu.SemaphoreType.DMA],
    )
    def disp_k(x_hbm, src_hbm, dst_hbm, out_hbm, src_v, dst_v, rows_v,
               gsem, wsem):
        wid = lax.axis_index("s") * NC + lax.axis_index("c")
        pltpu.sync_copy(src_hbm.at[wid], src_v)
        pltpu.sync_copy(dst_hbm.at[wid], dst_v)
        gh = [None, None]
        wh = [None, None]
        for i in range(nch):
            b = i % 2
            if i >= 2:
                wh[b].wait()
            gh[b] = pltpu.async_copy(x_hbm.at[src_v.at[i]], rows_v.at[b],
                                     gsem)
            if i >= 1:
                pb = (i - 1) % 2
                gh[pb].wait()
                wh[pb] = pltpu.async_copy(rows_v.at[pb],
                                          out_hbm.at[dst_v.at[i - 1]], wsem)
        lb = (nch - 1) % 2
        gh[lb].wait()
        pltpu.async_copy(rows_v.at[lb], out_hbm.at[dst_v.at[nch - 1]],
                         wsem).wait()
        if nch >= 2:
            wh[(nch - 2) % 2].wait()

    return disp_k(xf, src3, dst3)


def _unsort(out_sorted, idx3, nch, ch):
    per_w = nch * ch

    @functools.partial(
        pl.kernel,
        out_type=jax.ShapeDtypeStruct((A, D), jnp.float32),
        mesh=_sc_mesh(),
        scratch_types=[pltpu.VMEM((nch, ch), jnp.int32),
                       pltpu.VMEM((2, ch, D), jnp.float32),
                       pltpu.SemaphoreType.DMA,
                       pltpu.SemaphoreType.DMA],
    )
    def unsort_k(table_hbm, idx_hbm, out_hbm, idx_v, rows_v, gsem, wsem):
        wid = lax.axis_index("s") * NC + lax.axis_index("c")
        base = wid * per_w
        pltpu.sync_copy(idx_hbm.at[wid], idx_v)
        gh = [None, None]
        wh = [None, None]
        for i in range(nch):
            b = i % 2
            if i >= 2:
                wh[b].wait()
            gh[b] = pltpu.async_copy(table_hbm.at[idx_v.at[i]], rows_v.at[b],
                                     gsem)
            if i >= 1:
                pb = (i - 1) % 2
                gh[pb].wait()
                wh[pb] = pltpu.async_copy(
                    rows_v.at[pb],
                    out_hbm.at[pl.ds(base + (i - 1) * ch, ch)], wsem)
        lb = (nch - 1) % 2
        gh[lb].wait()
        pltpu.sync_copy(rows_v.at[lb],
                        out_hbm.at[pl.ds(base + (nch - 1) * ch, ch)])
        if nch >= 2:
            wh[(nch - 2) % 2].wait()

    return unsort_k(out_sorted, idx3)


def _grouped_body(eid_ref, nv_ref, x_ref, g_ref, u_ref, d_ref, o_ref):
    g = pl.program_id(0)

    @pl.when(nv_ref[g] > 0)
    def _():
        xg = x_ref[...].astype(jnp.bfloat16)
        wg = g_ref[0].astype(jnp.bfloat16)
        wu = u_ref[0].astype(jnp.bfloat16)
        wd = d_ref[0].astype(jnp.bfloat16)
        gg = lax.dot_general(xg, wg, _CONTRACT_MINOR,
                             preferred_element_type=jnp.float32)
        uu = lax.dot_general(xg, wu, _CONTRACT_MINOR,
                             preferred_element_type=jnp.float32)
        h = (gg * lax.logistic(gg) * uu).astype(jnp.bfloat16)
        o_ref[...] = lax.dot_general(h, wd, _CONTRACT_MINOR,
                                     preferred_element_type=jnp.float32)


def _grouped(x_sorted, gate_w, up_w, down_w, eid, nvalid):
    grid_spec = pltpu.PrefetchScalarGridSpec(
        num_scalar_prefetch=2,
        grid=(G,),
        in_specs=[
            pl.BlockSpec((BLK, D), lambda g, e, nv: (g, 0)),
            pl.BlockSpec((1, F, D), lambda g, e, nv: (e[g], 0, 0)),
            pl.BlockSpec((1, F, D), lambda g, e, nv: (e[g], 0, 0)),
            pl.BlockSpec((1, D, F), lambda g, e, nv: (e[g], 0, 0)),
        ],
        out_specs=pl.BlockSpec((BLK, D), lambda g, e, nv: (g, 0)),
    )
    return pl.pallas_call(
        _grouped_body,
        grid_spec=grid_spec,
        out_shape=jax.ShapeDtypeStruct((R, D), jnp.float32),
    )(eid, nvalid, x_sorted, gate_w, up_w, down_w)


def _shared_body(x_ref, g_ref, u_ref, d_ref, o_ref):
    xg = x_ref[...].astype(jnp.bfloat16)
    wg = g_ref[...].astype(jnp.bfloat16)
    wu = u_ref[...].astype(jnp.bfloat16)
    wd = d_ref[...].astype(jnp.bfloat16)
    gg = lax.dot_general(xg, wg, _CONTRACT_MINOR,
                         preferred_element_type=jnp.float32)
    uu = lax.dot_general(xg, wu, _CONTRACT_MINOR,
                         preferred_element_type=jnp.float32)
    h = (gg * lax.logistic(gg) * uu).astype(jnp.bfloat16)
    o_ref[...] = lax.dot_general(h, wd, _CONTRACT_MINOR,
                                 preferred_element_type=jnp.float32)


def _shared(xf, gate_w, up_w, down_w):
    bt = 512
    return pl.pallas_call(
        _shared_body,
        grid=(TOKS // bt,),
        in_specs=[pl.BlockSpec((bt, D), lambda g: (g, 0)),
                  pl.BlockSpec((F, D), lambda g: (0, 0)),
                  pl.BlockSpec((F, D), lambda g: (0, 0)),
                  pl.BlockSpec((D, F), lambda g: (0, 0))],
        out_specs=pl.BlockSpec((bt, D), lambda g: (g, 0)),
        out_shape=jax.ShapeDtypeStruct((TOKS, D), jnp.float32),
    )(xf, gate_w, up_w, down_w)


def _combine_body(sh_ref, p0_ref, p1_ref, tw_ref, o_ref):
    w0 = tw_ref[:, 0:1]
    w1 = tw_ref[:, 1:2]
    o_ref[...] = sh_ref[...] + w0 * p0_ref[...] + w1 * p1_ref[...]


def _combine(sh, pairs, top_w):
    bt = 1024
    nb = TOKS // bt
    return pl.pallas_call(
        _combine_body,
        grid=(nb,),
        in_specs=[pl.BlockSpec((bt, D), lambda g: (g, 0)),
                  pl.BlockSpec((bt, D), lambda g: (g, 0)),
                  pl.BlockSpec((bt, D), lambda g: (g + nb, 0)),
                  pl.BlockSpec((bt, K), lambda g: (g, 0))],
        out_specs=pl.BlockSpec((bt, D), lambda g: (g, 0)),
        out_shape=jax.ShapeDtypeStruct((TOKS, D), jnp.float32),
    )(sh, pairs, pairs, top_w)


def kernel(x, w_router, shared_gate, shared_up, shared_down,
           experts_gate, experts_up, experts_down):
    b, s, d = x.shape
    xf = x.reshape(-1, d)

    sh = _shared(xf, shared_gate, shared_up, shared_down)

    top_i, top_w = _router(xf, w_router)

    ae = top_i.reshape(A)
    onehot = (ae[:, None] == jnp.arange(E, dtype=jnp.int32)[None, :])
    ranks = jnp.cumsum(onehot.astype(jnp.int32), axis=0)
    counts = ranks[-1]
    rank = jnp.take_along_axis(ranks, ae[:, None], axis=1)[:, 0] - 1
    padded = ((counts + BLK - 1) // BLK) * BLK
    ends = jnp.cumsum(padded).astype(jnp.int32)
    offs = ends - padded
    dest = (offs[ae] + rank).astype(jnp.int32)

    gstart = jnp.arange(G, dtype=jnp.int32) * BLK
    eid = jnp.sum((gstart[:, None] >= ends[None, :]).astype(jnp.int32),
                  axis=1)
    eid_c = jnp.minimum(eid, E - 1)
    nvalid = jnp.where(
        eid < E,
        jnp.clip(offs[eid_c] + counts[eid_c] - gstart, 0, BLK),
        0).astype(jnp.int32)

    nch1, ch1 = 8, A // NW // 8
    src3 = (jnp.arange(A, dtype=jnp.int32) // K).reshape(NW, nch1, ch1)
    dst3 = dest.reshape(NW, nch1, ch1)
    x_sorted = _dispatch(xf, src3, dst3, nch1, ch1)

    out_sorted = _grouped(x_sorted, experts_gate, experts_up, experts_down,
                          eid_c, nvalid)

    idx_sm = dest.reshape(TOKS, K).T.reshape(NW, nch1, ch1)
    pairs = _unsort(out_sorted, idx_sm, nch1, ch1)

    out = _combine(sh, pairs, top_w)
    return out.reshape(b, s, d)
